# baseline (speedup 1.0000x reference)
import functools

import jax
import jax.numpy as jnp
from jax import lax
from jax.experimental import pallas as pl
from jax.experimental.pallas import tpu as pltpu
from jax.experimental.pallas import tpu_sc as plsc

N = 10000
C = 128
HC = 64
E = 320000
NUM_TYPES = 64
DIM_TOK = 62
STEPS = 4
BLOCKS = 2
T = 3
EPS = 1e-5
BR = 1000
NB = N // BR
R3N = 3 * N

K = 80
CHUNKS = E // K
NTILES = 16
CPT = CHUNKS // NTILES
EPT = E // NTILES

def _mesh():
    return plsc.VectorSubcoreMesh(core_axis_name="c", subcore_axis_name="s")


def _sc_prologue_body(ei_hbm, et_hbm, idx_hbm, srcb, dstb, etb, obuf):
    c = lax.axis_index("c")
    tid = lax.axis_index("s")
    c3n = c * R3N
    base = tid * EPT
    pltpu.sync_copy(ei_hbm.at[pl.ds(base, EPT)], srcb)
    pltpu.sync_copy(ei_hbm.at[pl.ds(E + base, EPT)], dstb)
    pltpu.sync_copy(et_hbm.at[pl.ds(base, EPT)], etb)

    def body(i, _):
        for l in range(K // 16):
            off = i * K + l * 16
            s = srcb[pl.ds(off, 16)]
            d = dstb[pl.ds(off, 16)]
            t = etb[pl.ds(off, 16)]
            tn = t * N
            obuf[i, 0, pl.ds(l * 16, 16)] = tn + s + c3n
            obuf[i, 1, pl.ds(l * 16, 16)] = tn + d
        return 0

    lax.fori_loop(0, CPT, body, 0)
    pltpu.sync_copy(obuf, idx_hbm.at[c, tid])


def _sc_prologue(edge_index_flat, edge_type):
    f = pl.kernel(
        _sc_prologue_body,
        out_type=jax.ShapeDtypeStruct((2, NTILES, CPT, 2, K), jnp.int32),
        mesh=_mesh(),
        scratch_types=[
            pltpu.VMEM((EPT,), jnp.int32),
            pltpu.VMEM((EPT,), jnp.int32),
            pltpu.VMEM((EPT,), jnp.int32),
            pltpu.VMEM((CPT, 2, K), jnp.int32),
        ],
        compiler_params=pltpu.CompilerParams(use_tc_tiling_on_sc=False),
    )
    return f(edge_index_flat, edge_type)


def _sc_scatter_body(m2_hbm, idx_hbm, z_hbm, agg_hbm,
                     idxb, rows, acc, semi0, semi1, semi2, semi3, semg0, semg1):
    c = lax.axis_index("c")
    tid = lax.axis_index("s")
    semi = (semi0, semi1, semi2, semi3)
    semg = (semg0, semg1)
    rpt = 1880
    rlast = R3N - 15 * rpt

    @pl.when(tid < NTILES - 1)
    def _():
        pltpu.sync_copy(z_hbm.at[pl.ds(tid * rpt, rpt)],
                        acc.at[pl.ds(tid * rpt, rpt)])

    @pl.when(tid == NTILES - 1)
    def _():
        pltpu.sync_copy(z_hbm.at[pl.ds(15 * rpt, rlast)],
                        acc.at[pl.ds(15 * rpt, rlast)])

    plsc.subcore_barrier()

    def fire_idx(j, q):
        pltpu.async_copy(idx_hbm.at[c, tid, j], idxb.at[q], semi[q])

    def wait_idx(j, q):
        pltpu.make_async_copy(idx_hbm.at[c, tid, j], idxb.at[q], semi[q]).wait()

    def fire_gather(j, q, r):
        pltpu.async_copy(m2_hbm.at[idxb.at[q, 0]], rows.at[r], semg[r])

    def wait_gather(r):
        pltpu.make_async_copy(m2_hbm.at[idxb.at[0, 0]], rows.at[r], semg[r]).wait()

    for q in range(4):
        fire_idx(q, q)
    wait_idx(0, 0)
    fire_gather(0, 0, 0)
    wait_idx(1, 1)
    fire_gather(1, 1, 1)

    def body(i, _):
        for k in range(4):
            j = 4 * i + k
            r = k % 2
            wait_gather(r)
            pltpu.sync_copy(rows.at[r], acc.at[idxb.at[k, 1]], add=True)

            @pl.when(j + 4 < CPT)
            def _():
                fire_idx(j + 4, k)

            @pl.when(j + 2 < CPT)
            def _():
                wait_idx(j + 2, (k + 2) % 4)
                fire_gather(j + 2, (k + 2) % 4, r)

        return 0

    lax.fori_loop(0, CPT // 4, body, 0)
    wait_gather(0)
    pltpu.sync_copy(rows.at[0], acc.at[idxb.at[0, 1]], add=True)
    wait_gather(1)
    pltpu.sync_copy(rows.at[1], acc.at[idxb.at[1, 1]], add=True)

    plsc.subcore_barrier()

    @pl.when(tid < NTILES - 1)
    def _():
        pltpu.sync_copy(acc.at[pl.ds(tid * rpt, rpt)],
                        agg_hbm.at[pl.ds(c * R3N + tid * rpt, rpt)])

    @pl.when(tid == NTILES - 1)
    def _():
        pltpu.sync_copy(acc.at[pl.ds(15 * rpt, rlast)],
                        agg_hbm.at[pl.ds(c * R3N + 15 * rpt, rlast)])


def _sc_scatter(m2, idx, zeros_tbl):
    f = pl.kernel(
        _sc_scatter_body,
        out_type=jax.ShapeDtypeStruct((2 * R3N, HC), jnp.float32),
        mesh=_mesh(),
        scratch_types=[
            pltpu.VMEM((4, 2, K), jnp.int32),
            pltpu.VMEM((2, K, HC), jnp.float32),
            pltpu.VMEM_SHARED((R3N, HC), jnp.float32),
            pltpu.SemaphoreType.DMA,
            pltpu.SemaphoreType.DMA,
            pltpu.SemaphoreType.DMA,
            pltpu.SemaphoreType.DMA,
            pltpu.SemaphoreType.DMA,
            pltpu.SemaphoreType.DMA,
        ],
        compiler_params=pltpu.CompilerParams(use_tc_tiling_on_sc=False),
    )
    return f(m2, idx, zeros_tbl)


def _feats_body(xt_ref, xk_ref, xs_ref, h_ref):
    col = lax.broadcasted_iota(jnp.int32, (N, C), 1)
    t = xt_ref[:]
    k = jnp.clip(xk_ref[:], 0, DIM_TOK - 1)
    xs = xs_ref[:]
    h = jnp.where(col < NUM_TYPES, (col == t).astype(jnp.float32),
                  jnp.where(col < NUM_TYPES + DIM_TOK,
                            (col - NUM_TYPES == k).astype(jnp.float32), 0.0))
    h = jnp.where(col == C - 2, xs[:, 0:1], h)
    h = jnp.where(col == C - 1, xs[:, 1:2], h)
    h_ref[:] = h


def _feats(x_type, x_tok, x_small):
    return pl.pallas_call(
        _feats_body,
        out_shape=jax.ShapeDtypeStruct((N, C), jnp.float32),
    )(x_type, x_tok, x_small)


def _conv0_body(h_ref, w_ref, x_ref, m_ref):
    x = h_ref[:]
    x_ref[0] = x
    m = jnp.dot(x, w_ref[0], preferred_element_type=jnp.float32)
    m_ref[0, 0] = m[:, :HC]
    m_ref[1, 0] = m[:, HC:]


def _conv0(h, w):
    return pl.pallas_call(
        _conv0_body,
        grid=(T, NB),
        in_specs=[
            pl.BlockSpec((BR, C), lambda t, rb: (rb, 0)),
            pl.BlockSpec((1, C, C), lambda t, rb: (t, 0, 0)),
        ],
        out_specs=[
            pl.BlockSpec((1, BR, C), lambda t, rb: (t, rb, 0)),
            pl.BlockSpec((2, 1, BR, HC), lambda t, rb: (0, t, rb, 0)),
        ],
        out_shape=[
            jax.ShapeDtypeStruct((T, N, C), jnp.float32),
            jax.ShapeDtypeStruct((2, T, N, HC), jnp.float32),
        ],
    )(h, w)


def _gru_math(x_ref, agg_ref, wih_ref, whh_ref, bih_ref, bhh_ref):
    x = x_ref[0]
    a = jnp.concatenate([agg_ref[0, 0], agg_ref[1, 0]], axis=1)
    gi = lax.dot_general(a, wih_ref[0], (((1,), (1,)), ((), ())),
                         preferred_element_type=jnp.float32) + bih_ref[0]
    gh = lax.dot_general(x, whh_ref[0], (((1,), (1,)), ((), ())),
                         preferred_element_type=jnp.float32) + bhh_ref[0]
    r = jax.nn.sigmoid(gi[:, :C] + gh[:, :C])
    z = jax.nn.sigmoid(gi[:, C:2 * C] + gh[:, C:2 * C])
    n = jnp.tanh(gi[:, 2 * C:] + r * gh[:, 2 * C:])
    return (1.0 - z) * n + z * x


def _gru_step_body(x_ref, agg_ref, wih_ref, whh_ref, bih_ref, bhh_ref,
                   wn_ref, xo_ref, m_ref):
    xn = _gru_math(x_ref, agg_ref, wih_ref, whh_ref, bih_ref, bhh_ref)
    xo_ref[0] = xn
    m = jnp.dot(xn, wn_ref[0], preferred_element_type=jnp.float32)
    m_ref[0, 0] = m[:, :HC]
    m_ref[1, 0] = m[:, HC:]


def _gru_last_body(x_ref, agg_ref, wih_ref, whh_ref, bih_ref, bhh_ref,
                   xo_ref):
    xo_ref[0] = _gru_math(x_ref, agg_ref, wih_ref, whh_ref, bih_ref, bhh_ref)


_GRU_IN_SPECS = [
    pl.BlockSpec((1, BR, C), lambda t, rb: (t, rb, 0)),
    pl.BlockSpec((2, 1, BR, HC), lambda t, rb: (0, t, rb, 0)),
    pl.BlockSpec((1, 3 * C, C), lambda t, rb: (t, 0, 0)),
    pl.BlockSpec((1, 3 * C, C), lambda t, rb: (t, 0, 0)),
    pl.BlockSpec((1, 1, 3 * C), lambda t, rb: (t, 0, 0)),
    pl.BlockSpec((1, 1, 3 * C), lambda t, rb: (t, 0, 0)),
]


def _gru_step(x_all, agg, wih, whh, bih, bhh, wn):
    return pl.pallas_call(
        _gru_step_body,
        grid=(T, NB),
        in_specs=_GRU_IN_SPECS + [pl.BlockSpec((1, C, C), lambda t, rb: (t, 0, 0))],
        out_specs=[
            pl.BlockSpec((1, BR, C), lambda t, rb: (t, rb, 0)),
            pl.BlockSpec((2, 1, BR, HC), lambda t, rb: (0, t, rb, 0)),
        ],
        out_shape=[
            jax.ShapeDtypeStruct((T, N, C), jnp.float32),
            jax.ShapeDtypeStruct((2, T, N, HC), jnp.float32),
        ],
    )(x_all, agg, wih, whh, bih, bhh, wn)


def _gru_last(x_all, agg, wih, whh, bih, bhh):
    return pl.pallas_call(
        _gru_last_body,
        grid=(T, NB),
        in_specs=_GRU_IN_SPECS,
        out_specs=pl.BlockSpec((1, BR, C), lambda t, rb: (t, rb, 0)),
        out_shape=jax.ShapeDtypeStruct((T, N, C), jnp.float32),
    )(x_all, agg, wih, whh, bih, bhh)


def _blockend_body(h_ref, x_ref, g_ref, b_ref, o_ref):
    hh = h_ref[:] + x_ref[0] + x_ref[1] + x_ref[2]
    mu = jnp.mean(hh, axis=-1, keepdims=True)
    var = jnp.mean((hh - mu) ** 2, axis=-1, keepdims=True)
    hn = (hh - mu) * lax.rsqrt(var + EPS) * g_ref[:] + b_ref[:]
    o_ref[:] = jnp.maximum(hn, 0.0)


def _blockend(h, x_all, gamma, beta):
    return pl.pallas_call(
        _blockend_body,
        out_shape=jax.ShapeDtypeStruct((N, C), jnp.float32),
    )(h, x_all, gamma, beta)


def _head_body(h_ref, w1_ref, b1_ref, w2_ref, b2_ref, o_ref):
    hid = jnp.maximum(
        jnp.dot(h_ref[:], w1_ref[:], preferred_element_type=jnp.float32)
        + b1_ref[:], 0.0)
    o_ref[:] = jnp.dot(hid, w2_ref[:], preferred_element_type=jnp.float32) + b2_ref[:]


def _head(h, w1, b1, w2p, b2p):
    return pl.pallas_call(
        _head_body,
        out_shape=jax.ShapeDtypeStruct((N, C), jnp.float32),
    )(h, w1, b1, w2p, b2p)


def kernel(x_type, x_tok, x_small, edge_index, edge_type, batch,
           conv_weight, gru_wih, gru_whh, gru_bih, gru_bhh,
           ln_gamma, ln_beta, head_w1, head_b1, head_w2, head_b2):
    del batch

    h = _feats(x_type, x_tok, x_small)
    perm = jnp.argsort(edge_type * N + edge_index[0])
    ei_sorted = edge_index[:, perm].reshape(2 * E)
    eidx = _sc_prologue(ei_sorted, edge_type[perm])
    zeros_tbl = jnp.zeros((R3N, HC), jnp.float32)

    bih3 = gru_bih.reshape(BLOCKS, T, 1, 3 * C)
    bhh3 = gru_bhh.reshape(BLOCKS, T, 1, 3 * C)

    for b in range(BLOCKS):
        x_all, m = _conv0(h, conv_weight[b, :, 0])
        for s in range(STEPS):
            agg = _sc_scatter(m.reshape(2 * R3N, HC), eidx, zeros_tbl)
            agg = agg.reshape(2, T, N, HC)
            if s < STEPS - 1:
                x_all, m = _gru_step(x_all, agg, gru_wih[b], gru_whh[b],
                                     bih3[b], bhh3[b], conv_weight[b, :, s + 1])
            else:
                x_all = _gru_last(x_all, agg, gru_wih[b], gru_whh[b],
                                  bih3[b], bhh3[b])
        h = _blockend(h, x_all, ln_gamma[b].reshape(1, C),
                      ln_beta[b].reshape(1, C))

    w2p = jnp.zeros((C, C), jnp.float32).at[:, :2].set(head_w2)
    b2p = jnp.zeros((1, C), jnp.float32).at[0, :2].set(head_b2)
    out = _head(h, head_w1, head_b1.reshape(1, C), w2p, b2p)
    return out[:, :2]


# speedup vs baseline: 1.6023x; 1.6023x over previous
import functools

import jax
import jax.numpy as jnp
from jax import lax
from jax.experimental import pallas as pl
from jax.experimental.pallas import tpu as pltpu
from jax.experimental.pallas import tpu_sc as plsc

N = 10000
C = 128
HC = 64
E = 320000
NUM_TYPES = 64
DIM_TOK = 62
STEPS = 4
BLOCKS = 2
T = 3
EPS = 1e-5
BR = 1000
NB = N // BR
R3N = 3 * N

K = 80
CHUNKS = E // K
NTILES = 16
CPT = CHUNKS // NTILES
EPT = E // NTILES

def _mesh():
    return plsc.VectorSubcoreMesh(core_axis_name="c", subcore_axis_name="s")


def _sc_prologue_body(ei_hbm, et_hbm, idx_hbm, srcb, dstb, etb, obuf):
    c = lax.axis_index("c")
    tid = lax.axis_index("s")
    c3n = c * R3N
    base = tid * EPT
    pltpu.sync_copy(ei_hbm.at[pl.ds(base, EPT)], srcb)
    pltpu.sync_copy(ei_hbm.at[pl.ds(E + base, EPT)], dstb)
    pltpu.sync_copy(et_hbm.at[pl.ds(base, EPT)], etb)

    def body(i, _):
        for l in range(K // 16):
            off = i * K + l * 16
            s = srcb[pl.ds(off, 16)]
            d = dstb[pl.ds(off, 16)]
            t = etb[pl.ds(off, 16)]
            tn = t * N
            obuf[i, 0, pl.ds(l * 16, 16)] = tn + s + c3n
            obuf[i, 1, pl.ds(l * 16, 16)] = tn + d
        return 0

    lax.fori_loop(0, CPT, body, 0)
    pltpu.sync_copy(obuf, idx_hbm.at[c, tid])


def _sc_prologue(edge_index_flat, edge_type):
    f = pl.kernel(
        _sc_prologue_body,
        out_type=jax.ShapeDtypeStruct((2, NTILES, CPT, 2, K), jnp.int32),
        mesh=_mesh(),
        scratch_types=[
            pltpu.VMEM((EPT,), jnp.int32),
            pltpu.VMEM((EPT,), jnp.int32),
            pltpu.VMEM((EPT,), jnp.int32),
            pltpu.VMEM((CPT, 2, K), jnp.int32),
        ],
        compiler_params=pltpu.CompilerParams(use_tc_tiling_on_sc=False),
    )
    return f(edge_index_flat, edge_type)


def _sc_scatter_body(m2_hbm, idx_hbm, z_hbm, agg_hbm,
                     idxb, rows, acc, semi0, semi1, semi2, semi3, semg0, semg1,
                     semg2a, semg2b):
    c = lax.axis_index("c")
    tid = lax.axis_index("s")
    semi = (semi0, semi1, semi2, semi3)
    semg = (semg0, semg1)
    semg2 = (semg2a, semg2b)
    rpt = 1880
    rlast = R3N - 15 * rpt

    @pl.when(tid < NTILES - 1)
    def _():
        pltpu.sync_copy(z_hbm.at[pl.ds(tid * rpt, rpt)],
                        acc.at[pl.ds(tid * rpt, rpt)])

    @pl.when(tid == NTILES - 1)
    def _():
        pltpu.sync_copy(z_hbm.at[pl.ds(15 * rpt, rlast)],
                        acc.at[pl.ds(15 * rpt, rlast)])

    plsc.subcore_barrier()

    def fire_idx(j, q):
        pltpu.async_copy(idx_hbm.at[c, tid, j], idxb.at[q], semi[q])

    def wait_idx(j, q):
        pltpu.make_async_copy(idx_hbm.at[c, tid, j], idxb.at[q], semi[q]).wait()

    def fire_gather(j, q, r):
        pltpu.async_copy(m2_hbm.at[idxb.at[q, 0, pl.ds(0, K // 2)]],
                         rows.at[r, pl.ds(0, K // 2)], semg[r])
        pltpu.async_copy(m2_hbm.at[idxb.at[q, 0, pl.ds(K // 2, K // 2)]],
                         rows.at[r, pl.ds(K // 2, K // 2)], semg2[r])

    def wait_gather(r):
        pltpu.make_async_copy(m2_hbm.at[idxb.at[0, 0, pl.ds(0, K // 2)]],
                              rows.at[r, pl.ds(0, K // 2)], semg[r]).wait()
        pltpu.make_async_copy(m2_hbm.at[idxb.at[0, 0, pl.ds(0, K // 2)]],
                              rows.at[r, pl.ds(K // 2, K // 2)], semg2[r]).wait()

    for q in range(4):
        fire_idx(q, q)
    wait_idx(0, 0)
    fire_gather(0, 0, 0)
    wait_idx(1, 1)
    fire_gather(1, 1, 1)

    def body(i, _):
        for k in range(4):
            j = 4 * i + k
            r = k % 2
            wait_gather(r)
            pltpu.sync_copy(rows.at[r], acc.at[idxb.at[k, 1]], add=True)

            @pl.when(j + 4 < CPT)
            def _():
                fire_idx(j + 4, k)

            @pl.when(j + 2 < CPT)
            def _():
                wait_idx(j + 2, (k + 2) % 4)
                fire_gather(j + 2, (k + 2) % 4, r)

        return 0

    lax.fori_loop(0, CPT // 4, body, 0)
    wait_gather(0)
    pltpu.sync_copy(rows.at[0], acc.at[idxb.at[0, 1]], add=True)
    wait_gather(1)
    pltpu.sync_copy(rows.at[1], acc.at[idxb.at[1, 1]], add=True)

    plsc.subcore_barrier()

    @pl.when(tid < NTILES - 1)
    def _():
        pltpu.sync_copy(acc.at[pl.ds(tid * rpt, rpt)],
                        agg_hbm.at[pl.ds(c * R3N + tid * rpt, rpt)])

    @pl.when(tid == NTILES - 1)
    def _():
        pltpu.sync_copy(acc.at[pl.ds(15 * rpt, rlast)],
                        agg_hbm.at[pl.ds(c * R3N + 15 * rpt, rlast)])


def _sc_scatter(m2, idx, zeros_tbl):
    f = pl.kernel(
        _sc_scatter_body,
        out_type=jax.ShapeDtypeStruct((2 * R3N, HC), jnp.float32),
        mesh=_mesh(),
        scratch_types=[
            pltpu.VMEM((4, 2, K), jnp.int32),
            pltpu.VMEM((2, K, HC), jnp.float32),
            pltpu.VMEM_SHARED((R3N, HC), jnp.float32),
            pltpu.SemaphoreType.DMA,
            pltpu.SemaphoreType.DMA,
            pltpu.SemaphoreType.DMA,
            pltpu.SemaphoreType.DMA,
            pltpu.SemaphoreType.DMA,
            pltpu.SemaphoreType.DMA,
            pltpu.SemaphoreType.DMA,
            pltpu.SemaphoreType.DMA,
        ],
        compiler_params=pltpu.CompilerParams(use_tc_tiling_on_sc=False),
    )
    return f(m2, idx, zeros_tbl)


def _feats_body(xt_ref, xk_ref, xs_ref, h_ref):
    col = lax.broadcasted_iota(jnp.int32, (N, C), 1)
    t = xt_ref[:]
    k = jnp.clip(xk_ref[:], 0, DIM_TOK - 1)
    xs = xs_ref[:]
    h = jnp.where(col < NUM_TYPES, (col == t).astype(jnp.float32),
                  jnp.where(col < NUM_TYPES + DIM_TOK,
                            (col - NUM_TYPES == k).astype(jnp.float32), 0.0))
    h = jnp.where(col == C - 2, xs[:, 0:1], h)
    h = jnp.where(col == C - 1, xs[:, 1:2], h)
    h_ref[:] = h


def _feats(x_type, x_tok, x_small):
    return pl.pallas_call(
        _feats_body,
        out_shape=jax.ShapeDtypeStruct((N, C), jnp.float32),
    )(x_type, x_tok, x_small)


def _conv0_body(h_ref, w_ref, x_ref, m_ref):
    x = h_ref[:]
    x_ref[0] = x
    m = jnp.dot(x, w_ref[0], preferred_element_type=jnp.float32)
    m_ref[0, 0] = m[:, :HC]
    m_ref[1, 0] = m[:, HC:]


def _conv0(h, w):
    return pl.pallas_call(
        _conv0_body,
        grid=(T, NB),
        in_specs=[
            pl.BlockSpec((BR, C), lambda t, rb: (rb, 0)),
            pl.BlockSpec((1, C, C), lambda t, rb: (t, 0, 0)),
        ],
        out_specs=[
            pl.BlockSpec((1, BR, C), lambda t, rb: (t, rb, 0)),
            pl.BlockSpec((2, 1, BR, HC), lambda t, rb: (0, t, rb, 0)),
        ],
        out_shape=[
            jax.ShapeDtypeStruct((T, N, C), jnp.float32),
            jax.ShapeDtypeStruct((2, T, N, HC), jnp.float32),
        ],
    )(h, w)


def _gru_math(x_ref, agg_ref, wih_ref, whh_ref, bih_ref, bhh_ref):
    x = x_ref[0]
    a = jnp.concatenate([agg_ref[0, 0], agg_ref[1, 0]], axis=1)
    gi = lax.dot_general(a, wih_ref[0], (((1,), (1,)), ((), ())),
                         preferred_element_type=jnp.float32) + bih_ref[0]
    gh = lax.dot_general(x, whh_ref[0], (((1,), (1,)), ((), ())),
                         preferred_element_type=jnp.float32) + bhh_ref[0]
    r = jax.nn.sigmoid(gi[:, :C] + gh[:, :C])
    z = jax.nn.sigmoid(gi[:, C:2 * C] + gh[:, C:2 * C])
    n = jnp.tanh(gi[:, 2 * C:] + r * gh[:, 2 * C:])
    return (1.0 - z) * n + z * x


def _gru_step_body(x_ref, agg_ref, wih_ref, whh_ref, bih_ref, bhh_ref,
                   wn_ref, xo_ref, m_ref):
    xn = _gru_math(x_ref, agg_ref, wih_ref, whh_ref, bih_ref, bhh_ref)
    xo_ref[0] = xn
    m = jnp.dot(xn, wn_ref[0], preferred_element_type=jnp.float32)
    m_ref[0, 0] = m[:, :HC]
    m_ref[1, 0] = m[:, HC:]


def _gru_last_body(x_ref, agg_ref, wih_ref, whh_ref, bih_ref, bhh_ref,
                   xo_ref):
    xo_ref[0] = _gru_math(x_ref, agg_ref, wih_ref, whh_ref, bih_ref, bhh_ref)


_GRU_IN_SPECS = [
    pl.BlockSpec((1, BR, C), lambda t, rb: (t, rb, 0)),
    pl.BlockSpec((2, 1, BR, HC), lambda t, rb: (0, t, rb, 0)),
    pl.BlockSpec((1, 3 * C, C), lambda t, rb: (t, 0, 0)),
    pl.BlockSpec((1, 3 * C, C), lambda t, rb: (t, 0, 0)),
    pl.BlockSpec((1, 1, 3 * C), lambda t, rb: (t, 0, 0)),
    pl.BlockSpec((1, 1, 3 * C), lambda t, rb: (t, 0, 0)),
]


def _gru_step(x_all, agg, wih, whh, bih, bhh, wn):
    return pl.pallas_call(
        _gru_step_body,
        grid=(T, NB),
        in_specs=_GRU_IN_SPECS + [pl.BlockSpec((1, C, C), lambda t, rb: (t, 0, 0))],
        out_specs=[
            pl.BlockSpec((1, BR, C), lambda t, rb: (t, rb, 0)),
            pl.BlockSpec((2, 1, BR, HC), lambda t, rb: (0, t, rb, 0)),
        ],
        out_shape=[
            jax.ShapeDtypeStruct((T, N, C), jnp.float32),
            jax.ShapeDtypeStruct((2, T, N, HC), jnp.float32),
        ],
    )(x_all, agg, wih, whh, bih, bhh, wn)


def _gru_last(x_all, agg, wih, whh, bih, bhh):
    return pl.pallas_call(
        _gru_last_body,
        grid=(T, NB),
        in_specs=_GRU_IN_SPECS,
        out_specs=pl.BlockSpec((1, BR, C), lambda t, rb: (t, rb, 0)),
        out_shape=jax.ShapeDtypeStruct((T, N, C), jnp.float32),
    )(x_all, agg, wih, whh, bih, bhh)


def _blockend_body(h_ref, x_ref, g_ref, b_ref, o_ref):
    hh = h_ref[:] + x_ref[0] + x_ref[1] + x_ref[2]
    mu = jnp.mean(hh, axis=-1, keepdims=True)
    var = jnp.mean((hh - mu) ** 2, axis=-1, keepdims=True)
    hn = (hh - mu) * lax.rsqrt(var + EPS) * g_ref[:] + b_ref[:]
    o_ref[:] = jnp.maximum(hn, 0.0)


def _blockend(h, x_all, gamma, beta):
    return pl.pallas_call(
        _blockend_body,
        out_shape=jax.ShapeDtypeStruct((N, C), jnp.float32),
    )(h, x_all, gamma, beta)


def _head_body(h_ref, w1_ref, b1_ref, w2_ref, b2_ref, o_ref):
    hid = jnp.maximum(
        jnp.dot(h_ref[:], w1_ref[:], preferred_element_type=jnp.float32)
        + b1_ref[:], 0.0)
    o_ref[:] = jnp.dot(hid, w2_ref[:], preferred_element_type=jnp.float32) + b2_ref[:]


def _head(h, w1, b1, w2p, b2p):
    return pl.pallas_call(
        _head_body,
        out_shape=jax.ShapeDtypeStruct((N, C), jnp.float32),
    )(h, w1, b1, w2p, b2p)


def kernel(x_type, x_tok, x_small, edge_index, edge_type, batch,
           conv_weight, gru_wih, gru_whh, gru_bih, gru_bhh,
           ln_gamma, ln_beta, head_w1, head_b1, head_w2, head_b2):
    del batch

    h = _feats(x_type, x_tok, x_small)
    eidx = _sc_prologue(edge_index.reshape(2 * E), edge_type)
    zeros_tbl = jnp.zeros((R3N, HC), jnp.float32)

    bih3 = gru_bih.reshape(BLOCKS, T, 1, 3 * C)
    bhh3 = gru_bhh.reshape(BLOCKS, T, 1, 3 * C)

    for b in range(BLOCKS):
        x_all, m = _conv0(h, conv_weight[b, :, 0])
        for s in range(STEPS):
            agg = _sc_scatter(m.reshape(2 * R3N, HC), eidx, zeros_tbl)
            agg = agg.reshape(2, T, N, HC)
            if s < STEPS - 1:
                x_all, m = _gru_step(x_all, agg, gru_wih[b], gru_whh[b],
                                     bih3[b], bhh3[b], conv_weight[b, :, s + 1])
            else:
                x_all = _gru_last(x_all, agg, gru_wih[b], gru_whh[b],
                                  bih3[b], bhh3[b])
        h = _blockend(h, x_all, ln_gamma[b].reshape(1, C),
                      ln_beta[b].reshape(1, C))

    w2p = jnp.zeros((C, C), jnp.float32).at[:, :2].set(head_w2)
    b2p = jnp.zeros((1, C), jnp.float32).at[0, :2].set(head_b2)
    out = _head(h, head_w1, head_b1.reshape(1, C), w2p, b2p)
    return out[:, :2]


# speedup vs baseline: 1.6879x; 1.0534x over previous
import functools

import jax
import jax.numpy as jnp
from jax import lax
from jax.experimental import pallas as pl
from jax.experimental.pallas import tpu as pltpu
from jax.experimental.pallas import tpu_sc as plsc

N = 10000
C = 128
HC = 64
E = 320000
NUM_TYPES = 64
DIM_TOK = 62
STEPS = 4
BLOCKS = 2
T = 3
EPS = 1e-5
BR = 2000
NB = N // BR
R3N = 3 * N

K = 80
CHUNKS = E // K
NTILES = 16
CPT = CHUNKS // NTILES
EPT = E // NTILES

def _mesh():
    return plsc.VectorSubcoreMesh(core_axis_name="c", subcore_axis_name="s")


def _sc_prologue_body(ei_hbm, et_hbm, idx_hbm, srcb, dstb, etb, obuf):
    c = lax.axis_index("c")
    tid = lax.axis_index("s")
    c3n = c * R3N
    base = tid * EPT
    pltpu.sync_copy(ei_hbm.at[pl.ds(base, EPT)], srcb)
    pltpu.sync_copy(ei_hbm.at[pl.ds(E + base, EPT)], dstb)
    pltpu.sync_copy(et_hbm.at[pl.ds(base, EPT)], etb)

    def body(i, _):
        for l in range(K // 16):
            off = i * K + l * 16
            s = srcb[pl.ds(off, 16)]
            d = dstb[pl.ds(off, 16)]
            t = etb[pl.ds(off, 16)]
            tn = t * N
            obuf[i, 0, pl.ds(l * 16, 16)] = tn + s + c3n
            obuf[i, 1, pl.ds(l * 16, 16)] = tn + d
        return 0

    lax.fori_loop(0, CPT, body, 0)
    pltpu.sync_copy(obuf, idx_hbm.at[c, tid])


def _sc_prologue(edge_index_flat, edge_type):
    f = pl.kernel(
        _sc_prologue_body,
        out_type=jax.ShapeDtypeStruct((2, NTILES, CPT, 2, K), jnp.int32),
        mesh=_mesh(),
        scratch_types=[
            pltpu.VMEM((EPT,), jnp.int32),
            pltpu.VMEM((EPT,), jnp.int32),
            pltpu.VMEM((EPT,), jnp.int32),
            pltpu.VMEM((CPT, 2, K), jnp.int32),
        ],
        compiler_params=pltpu.CompilerParams(use_tc_tiling_on_sc=False),
    )
    return f(edge_index_flat, edge_type)


def _sc_scatter_body(m2_hbm, idx_hbm, z_hbm, agg_hbm,
                     idxb, rows, acc, semi0, semi1, semi2, semi3, semg0, semg1,
                     semg2a, semg2b):
    c = lax.axis_index("c")
    tid = lax.axis_index("s")
    semi = (semi0, semi1, semi2, semi3)
    semg = (semg0, semg1)
    semg2 = (semg2a, semg2b)
    rpt = 1880
    rlast = R3N - 15 * rpt

    @pl.when(tid < NTILES - 1)
    def _():
        pltpu.sync_copy(z_hbm.at[pl.ds(tid * rpt, rpt)],
                        acc.at[pl.ds(tid * rpt, rpt)])

    @pl.when(tid == NTILES - 1)
    def _():
        pltpu.sync_copy(z_hbm.at[pl.ds(15 * rpt, rlast)],
                        acc.at[pl.ds(15 * rpt, rlast)])

    plsc.subcore_barrier()

    def fire_idx(j, q):
        pltpu.async_copy(idx_hbm.at[c, tid, j], idxb.at[q], semi[q])

    def wait_idx(j, q):
        pltpu.make_async_copy(idx_hbm.at[c, tid, j], idxb.at[q], semi[q]).wait()

    def fire_gather(j, q, r):
        pltpu.async_copy(m2_hbm.at[idxb.at[q, 0, pl.ds(0, K // 2)]],
                         rows.at[r, pl.ds(0, K // 2)], semg[r])
        pltpu.async_copy(m2_hbm.at[idxb.at[q, 0, pl.ds(K // 2, K // 2)]],
                         rows.at[r, pl.ds(K // 2, K // 2)], semg2[r])

    def wait_gather(r):
        pltpu.make_async_copy(m2_hbm.at[idxb.at[0, 0, pl.ds(0, K // 2)]],
                              rows.at[r, pl.ds(0, K // 2)], semg[r]).wait()
        pltpu.make_async_copy(m2_hbm.at[idxb.at[0, 0, pl.ds(0, K // 2)]],
                              rows.at[r, pl.ds(K // 2, K // 2)], semg2[r]).wait()

    for q in range(4):
        fire_idx(q, q)
    wait_idx(0, 0)
    fire_gather(0, 0, 0)
    wait_idx(1, 1)
    fire_gather(1, 1, 1)

    def body(i, _):
        for k in range(4):
            j = 4 * i + k
            r = k % 2
            wait_gather(r)
            pltpu.sync_copy(rows.at[r], acc.at[idxb.at[k, 1]], add=True)

            @pl.when(j + 4 < CPT)
            def _():
                fire_idx(j + 4, k)

            @pl.when(j + 2 < CPT)
            def _():
                wait_idx(j + 2, (k + 2) % 4)
                fire_gather(j + 2, (k + 2) % 4, r)

        return 0

    lax.fori_loop(0, CPT // 4, body, 0)
    wait_gather(0)
    pltpu.sync_copy(rows.at[0], acc.at[idxb.at[0, 1]], add=True)
    wait_gather(1)
    pltpu.sync_copy(rows.at[1], acc.at[idxb.at[1, 1]], add=True)

    plsc.subcore_barrier()

    @pl.when(tid < NTILES - 1)
    def _():
        pltpu.sync_copy(acc.at[pl.ds(tid * rpt, rpt)],
                        agg_hbm.at[pl.ds(c * R3N + tid * rpt, rpt)])

    @pl.when(tid == NTILES - 1)
    def _():
        pltpu.sync_copy(acc.at[pl.ds(15 * rpt, rlast)],
                        agg_hbm.at[pl.ds(c * R3N + 15 * rpt, rlast)])


def _sc_scatter(m2, idx, zeros_tbl):
    f = pl.kernel(
        _sc_scatter_body,
        out_type=jax.ShapeDtypeStruct((2 * R3N, HC), jnp.float32),
        mesh=_mesh(),
        scratch_types=[
            pltpu.VMEM((4, 2, K), jnp.int32),
            pltpu.VMEM((2, K, HC), jnp.float32),
            pltpu.VMEM_SHARED((R3N, HC), jnp.float32),
            pltpu.SemaphoreType.DMA,
            pltpu.SemaphoreType.DMA,
            pltpu.SemaphoreType.DMA,
            pltpu.SemaphoreType.DMA,
            pltpu.SemaphoreType.DMA,
            pltpu.SemaphoreType.DMA,
            pltpu.SemaphoreType.DMA,
            pltpu.SemaphoreType.DMA,
        ],
        compiler_params=pltpu.CompilerParams(use_tc_tiling_on_sc=False),
    )
    return f(m2, idx, zeros_tbl)


def _feats_body(xt_ref, xk_ref, xs_ref, h_ref):
    col = lax.broadcasted_iota(jnp.int32, (N, C), 1)
    t = xt_ref[:]
    k = jnp.clip(xk_ref[:], 0, DIM_TOK - 1)
    xs = xs_ref[:]
    h = jnp.where(col < NUM_TYPES, (col == t).astype(jnp.float32),
                  jnp.where(col < NUM_TYPES + DIM_TOK,
                            (col - NUM_TYPES == k).astype(jnp.float32), 0.0))
    h = jnp.where(col == C - 2, xs[:, 0:1], h)
    h = jnp.where(col == C - 1, xs[:, 1:2], h)
    h_ref[:] = h


def _feats(x_type, x_tok, x_small):
    return pl.pallas_call(
        _feats_body,
        out_shape=jax.ShapeDtypeStruct((N, C), jnp.float32),
    )(x_type, x_tok, x_small)


def _conv0_body(h_ref, w_ref, x_ref, m_ref):
    x = h_ref[:]
    x_ref[0] = x
    m = jnp.dot(x, w_ref[0], preferred_element_type=jnp.float32)
    m_ref[0, 0] = m[:, :HC]
    m_ref[1, 0] = m[:, HC:]


def _conv0(h, w):
    return pl.pallas_call(
        _conv0_body,
        grid=(T, NB),
        in_specs=[
            pl.BlockSpec((BR, C), lambda t, rb: (rb, 0)),
            pl.BlockSpec((1, C, C), lambda t, rb: (t, 0, 0)),
        ],
        out_specs=[
            pl.BlockSpec((1, BR, C), lambda t, rb: (t, rb, 0)),
            pl.BlockSpec((2, 1, BR, HC), lambda t, rb: (0, t, rb, 0)),
        ],
        out_shape=[
            jax.ShapeDtypeStruct((T, N, C), jnp.float32),
            jax.ShapeDtypeStruct((2, T, N, HC), jnp.float32),
        ],
    )(h, w)


def _gru_math(x_ref, agg_ref, wih_ref, whh_ref, bih_ref, bhh_ref):
    x = x_ref[0]
    a = jnp.concatenate([agg_ref[0, 0], agg_ref[1, 0]], axis=1)
    gi = lax.dot_general(a, wih_ref[0], (((1,), (1,)), ((), ())),
                         preferred_element_type=jnp.float32) + bih_ref[0]
    gh = lax.dot_general(x, whh_ref[0], (((1,), (1,)), ((), ())),
                         preferred_element_type=jnp.float32) + bhh_ref[0]
    r = jax.nn.sigmoid(gi[:, :C] + gh[:, :C])
    z = jax.nn.sigmoid(gi[:, C:2 * C] + gh[:, C:2 * C])
    n = jnp.tanh(gi[:, 2 * C:] + r * gh[:, 2 * C:])
    return (1.0 - z) * n + z * x


def _gru_step_body(x_ref, agg_ref, wih_ref, whh_ref, bih_ref, bhh_ref,
                   wn_ref, xo_ref, m_ref):
    xn = _gru_math(x_ref, agg_ref, wih_ref, whh_ref, bih_ref, bhh_ref)
    xo_ref[0] = xn
    m = jnp.dot(xn, wn_ref[0], preferred_element_type=jnp.float32)
    m_ref[0, 0] = m[:, :HC]
    m_ref[1, 0] = m[:, HC:]


def _gru_last_body(x_ref, agg_ref, wih_ref, whh_ref, bih_ref, bhh_ref,
                   xo_ref):
    xo_ref[0] = _gru_math(x_ref, agg_ref, wih_ref, whh_ref, bih_ref, bhh_ref)


_GRU_IN_SPECS = [
    pl.BlockSpec((1, BR, C), lambda t, rb: (t, rb, 0)),
    pl.BlockSpec((2, 1, BR, HC), lambda t, rb: (0, t, rb, 0)),
    pl.BlockSpec((1, 3 * C, C), lambda t, rb: (t, 0, 0)),
    pl.BlockSpec((1, 3 * C, C), lambda t, rb: (t, 0, 0)),
    pl.BlockSpec((1, 1, 3 * C), lambda t, rb: (t, 0, 0)),
    pl.BlockSpec((1, 1, 3 * C), lambda t, rb: (t, 0, 0)),
]


def _gru_step(x_all, agg, wih, whh, bih, bhh, wn):
    return pl.pallas_call(
        _gru_step_body,
        grid=(T, NB),
        in_specs=_GRU_IN_SPECS + [pl.BlockSpec((1, C, C), lambda t, rb: (t, 0, 0))],
        out_specs=[
            pl.BlockSpec((1, BR, C), lambda t, rb: (t, rb, 0)),
            pl.BlockSpec((2, 1, BR, HC), lambda t, rb: (0, t, rb, 0)),
        ],
        out_shape=[
            jax.ShapeDtypeStruct((T, N, C), jnp.float32),
            jax.ShapeDtypeStruct((2, T, N, HC), jnp.float32),
        ],
    )(x_all, agg, wih, whh, bih, bhh, wn)


def _gru_last(x_all, agg, wih, whh, bih, bhh):
    return pl.pallas_call(
        _gru_last_body,
        grid=(T, NB),
        in_specs=_GRU_IN_SPECS,
        out_specs=pl.BlockSpec((1, BR, C), lambda t, rb: (t, rb, 0)),
        out_shape=jax.ShapeDtypeStruct((T, N, C), jnp.float32),
    )(x_all, agg, wih, whh, bih, bhh)


def _blockend_body(h_ref, x_ref, g_ref, b_ref, o_ref):
    hh = h_ref[:] + x_ref[0] + x_ref[1] + x_ref[2]
    mu = jnp.mean(hh, axis=-1, keepdims=True)
    var = jnp.mean((hh - mu) ** 2, axis=-1, keepdims=True)
    hn = (hh - mu) * lax.rsqrt(var + EPS) * g_ref[:] + b_ref[:]
    o_ref[:] = jnp.maximum(hn, 0.0)


def _blockend(h, x_all, gamma, beta):
    return pl.pallas_call(
        _blockend_body,
        out_shape=jax.ShapeDtypeStruct((N, C), jnp.float32),
    )(h, x_all, gamma, beta)


def _head_body(h_ref, w1_ref, b1_ref, w2_ref, b2_ref, o_ref):
    hid = jnp.maximum(
        jnp.dot(h_ref[:], w1_ref[:], preferred_element_type=jnp.float32)
        + b1_ref[:], 0.0)
    o_ref[:] = jnp.dot(hid, w2_ref[:], preferred_element_type=jnp.float32) + b2_ref[:]


def _head(h, w1, b1, w2p, b2p):
    return pl.pallas_call(
        _head_body,
        out_shape=jax.ShapeDtypeStruct((N, C), jnp.float32),
    )(h, w1, b1, w2p, b2p)


def kernel(x_type, x_tok, x_small, edge_index, edge_type, batch,
           conv_weight, gru_wih, gru_whh, gru_bih, gru_bhh,
           ln_gamma, ln_beta, head_w1, head_b1, head_w2, head_b2):
    del batch

    h = _feats(x_type, x_tok, x_small)
    eidx = _sc_prologue(edge_index.reshape(2 * E), edge_type)
    zeros_tbl = jnp.zeros((R3N, HC), jnp.float32)

    bih3 = gru_bih.reshape(BLOCKS, T, 1, 3 * C)
    bhh3 = gru_bhh.reshape(BLOCKS, T, 1, 3 * C)

    for b in range(BLOCKS):
        x_all, m = _conv0(h, conv_weight[b, :, 0])
        for s in range(STEPS):
            agg = _sc_scatter(m.reshape(2 * R3N, HC), eidx, zeros_tbl)
            agg = agg.reshape(2, T, N, HC)
            if s < STEPS - 1:
                x_all, m = _gru_step(x_all, agg, gru_wih[b], gru_whh[b],
                                     bih3[b], bhh3[b], conv_weight[b, :, s + 1])
            else:
                x_all = _gru_last(x_all, agg, gru_wih[b], gru_whh[b],
                                  bih3[b], bhh3[b])
        h = _blockend(h, x_all, ln_gamma[b].reshape(1, C),
                      ln_beta[b].reshape(1, C))

    w2p = jnp.zeros((C, C), jnp.float32).at[:, :2].set(head_w2)
    b2p = jnp.zeros((1, C), jnp.float32).at[0, :2].set(head_b2)
    out = _head(h, head_w1, head_b1.reshape(1, C), w2p, b2p)
    return out[:, :2]


# speedup vs baseline: 1.6884x; 1.0003x over previous
import functools

import jax
import jax.numpy as jnp
from jax import lax
from jax.experimental import pallas as pl
from jax.experimental.pallas import tpu as pltpu
from jax.experimental.pallas import tpu_sc as plsc

N = 10000
C = 128
HC = 64
E = 320000
NUM_TYPES = 64
DIM_TOK = 62
STEPS = 4
BLOCKS = 2
T = 3
EPS = 1e-5
BR = 2000
NB = N // BR
R3N = 3 * N

K = 80
CHUNKS = E // K
NTILES = 16
CPT = CHUNKS // NTILES
EPT = E // NTILES

def _mesh():
    return plsc.VectorSubcoreMesh(core_axis_name="c", subcore_axis_name="s")


def _sc_prologue_body(ei_hbm, et_hbm, idx_hbm, srcb, dstb, etb, obuf):
    c = lax.axis_index("c")
    tid = lax.axis_index("s")
    c3n = c * R3N
    base = tid * EPT
    pltpu.sync_copy(ei_hbm.at[pl.ds(base, EPT)], srcb)
    pltpu.sync_copy(ei_hbm.at[pl.ds(E + base, EPT)], dstb)
    pltpu.sync_copy(et_hbm.at[pl.ds(base, EPT)], etb)

    def body(i, _):
        for l in range(K // 16):
            off = i * K + l * 16
            s = srcb[pl.ds(off, 16)]
            d = dstb[pl.ds(off, 16)]
            t = etb[pl.ds(off, 16)]
            tn = t * N
            obuf[i, 0, pl.ds(l * 16, 16)] = tn + s + c3n
            obuf[i, 1, pl.ds(l * 16, 16)] = tn + d
        return 0

    lax.fori_loop(0, CPT, body, 0)
    pltpu.sync_copy(obuf, idx_hbm.at[c, tid])


def _sc_prologue(edge_index_flat, edge_type):
    f = pl.kernel(
        _sc_prologue_body,
        out_type=jax.ShapeDtypeStruct((2, NTILES, CPT, 2, K), jnp.int32),
        mesh=_mesh(),
        scratch_types=[
            pltpu.VMEM((EPT,), jnp.int32),
            pltpu.VMEM((EPT,), jnp.int32),
            pltpu.VMEM((EPT,), jnp.int32),
            pltpu.VMEM((CPT, 2, K), jnp.int32),
        ],
        compiler_params=pltpu.CompilerParams(use_tc_tiling_on_sc=False, skip_device_barrier=True),
    )
    return f(edge_index_flat, edge_type)


def _sc_scatter_body(m2_hbm, idx_hbm, z_hbm, agg_hbm,
                     idxb, rows, acc, semi0, semi1, semi2, semi3, semg0, semg1,
                     semg2a, semg2b):
    c = lax.axis_index("c")
    tid = lax.axis_index("s")
    semi = (semi0, semi1, semi2, semi3)
    semg = (semg0, semg1)
    semg2 = (semg2a, semg2b)
    rpt = 1880
    rlast = R3N - 15 * rpt

    @pl.when(tid < NTILES - 1)
    def _():
        pltpu.sync_copy(z_hbm.at[pl.ds(tid * rpt, rpt)],
                        acc.at[pl.ds(tid * rpt, rpt)])

    @pl.when(tid == NTILES - 1)
    def _():
        pltpu.sync_copy(z_hbm.at[pl.ds(15 * rpt, rlast)],
                        acc.at[pl.ds(15 * rpt, rlast)])

    plsc.subcore_barrier()

    def fire_idx(j, q):
        pltpu.async_copy(idx_hbm.at[c, tid, j], idxb.at[q], semi[q])

    def wait_idx(j, q):
        pltpu.make_async_copy(idx_hbm.at[c, tid, j], idxb.at[q], semi[q]).wait()

    def fire_gather(j, q, r):
        pltpu.async_copy(m2_hbm.at[idxb.at[q, 0, pl.ds(0, K // 2)]],
                         rows.at[r, pl.ds(0, K // 2)], semg[r])
        pltpu.async_copy(m2_hbm.at[idxb.at[q, 0, pl.ds(K // 2, K // 2)]],
                         rows.at[r, pl.ds(K // 2, K // 2)], semg2[r])

    def wait_gather(r):
        pltpu.make_async_copy(m2_hbm.at[idxb.at[0, 0, pl.ds(0, K // 2)]],
                              rows.at[r, pl.ds(0, K // 2)], semg[r]).wait()
        pltpu.make_async_copy(m2_hbm.at[idxb.at[0, 0, pl.ds(0, K // 2)]],
                              rows.at[r, pl.ds(K // 2, K // 2)], semg2[r]).wait()

    for q in range(4):
        fire_idx(q, q)
    wait_idx(0, 0)
    fire_gather(0, 0, 0)
    wait_idx(1, 1)
    fire_gather(1, 1, 1)

    def body(i, _):
        for k in range(4):
            j = 4 * i + k
            r = k % 2
            wait_gather(r)
            pltpu.sync_copy(rows.at[r], acc.at[idxb.at[k, 1]], add=True)

            @pl.when(j + 4 < CPT)
            def _():
                fire_idx(j + 4, k)

            @pl.when(j + 2 < CPT)
            def _():
                wait_idx(j + 2, (k + 2) % 4)
                fire_gather(j + 2, (k + 2) % 4, r)

        return 0

    lax.fori_loop(0, CPT // 4, body, 0)
    wait_gather(0)
    pltpu.sync_copy(rows.at[0], acc.at[idxb.at[0, 1]], add=True)
    wait_gather(1)
    pltpu.sync_copy(rows.at[1], acc.at[idxb.at[1, 1]], add=True)

    plsc.subcore_barrier()

    @pl.when(tid < NTILES - 1)
    def _():
        pltpu.sync_copy(acc.at[pl.ds(tid * rpt, rpt)],
                        agg_hbm.at[pl.ds(c * R3N + tid * rpt, rpt)])

    @pl.when(tid == NTILES - 1)
    def _():
        pltpu.sync_copy(acc.at[pl.ds(15 * rpt, rlast)],
                        agg_hbm.at[pl.ds(c * R3N + 15 * rpt, rlast)])


def _sc_scatter(m2, idx, zeros_tbl):
    f = pl.kernel(
        _sc_scatter_body,
        out_type=jax.ShapeDtypeStruct((2 * R3N, HC), jnp.float32),
        mesh=_mesh(),
        scratch_types=[
            pltpu.VMEM((4, 2, K), jnp.int32),
            pltpu.VMEM((2, K, HC), jnp.float32),
            pltpu.VMEM_SHARED((R3N, HC), jnp.float32),
            pltpu.SemaphoreType.DMA,
            pltpu.SemaphoreType.DMA,
            pltpu.SemaphoreType.DMA,
            pltpu.SemaphoreType.DMA,
            pltpu.SemaphoreType.DMA,
            pltpu.SemaphoreType.DMA,
            pltpu.SemaphoreType.DMA,
            pltpu.SemaphoreType.DMA,
        ],
        compiler_params=pltpu.CompilerParams(use_tc_tiling_on_sc=False, skip_device_barrier=True),
    )
    return f(m2, idx, zeros_tbl)


def _feats_body(xt_ref, xk_ref, xs_ref, h_ref):
    col = lax.broadcasted_iota(jnp.int32, (N, C), 1)
    t = xt_ref[:]
    k = jnp.clip(xk_ref[:], 0, DIM_TOK - 1)
    xs = xs_ref[:]
    h = jnp.where(col < NUM_TYPES, (col == t).astype(jnp.float32),
                  jnp.where(col < NUM_TYPES + DIM_TOK,
                            (col - NUM_TYPES == k).astype(jnp.float32), 0.0))
    h = jnp.where(col == C - 2, xs[:, 0:1], h)
    h = jnp.where(col == C - 1, xs[:, 1:2], h)
    h_ref[:] = h


def _feats(x_type, x_tok, x_small):
    return pl.pallas_call(
        _feats_body,
        out_shape=jax.ShapeDtypeStruct((N, C), jnp.float32),
    )(x_type, x_tok, x_small)


def _conv0_body(h_ref, w_ref, x_ref, m_ref):
    x = h_ref[:]
    x_ref[0] = x
    m = jnp.dot(x, w_ref[0], preferred_element_type=jnp.float32)
    m_ref[0, 0] = m[:, :HC]
    m_ref[1, 0] = m[:, HC:]


def _conv0(h, w):
    return pl.pallas_call(
        _conv0_body,
        grid=(T, NB),
        in_specs=[
            pl.BlockSpec((BR, C), lambda t, rb: (rb, 0)),
            pl.BlockSpec((1, C, C), lambda t, rb: (t, 0, 0)),
        ],
        out_specs=[
            pl.BlockSpec((1, BR, C), lambda t, rb: (t, rb, 0)),
            pl.BlockSpec((2, 1, BR, HC), lambda t, rb: (0, t, rb, 0)),
        ],
        out_shape=[
            jax.ShapeDtypeStruct((T, N, C), jnp.float32),
            jax.ShapeDtypeStruct((2, T, N, HC), jnp.float32),
        ],
    )(h, w)


def _gru_math(x_ref, agg_ref, wih_ref, whh_ref, bih_ref, bhh_ref):
    x = x_ref[0]
    a = jnp.concatenate([agg_ref[0, 0], agg_ref[1, 0]], axis=1)
    gi = lax.dot_general(a, wih_ref[0], (((1,), (1,)), ((), ())),
                         preferred_element_type=jnp.float32) + bih_ref[0]
    gh = lax.dot_general(x, whh_ref[0], (((1,), (1,)), ((), ())),
                         preferred_element_type=jnp.float32) + bhh_ref[0]
    r = jax.nn.sigmoid(gi[:, :C] + gh[:, :C])
    z = jax.nn.sigmoid(gi[:, C:2 * C] + gh[:, C:2 * C])
    n = jnp.tanh(gi[:, 2 * C:] + r * gh[:, 2 * C:])
    return (1.0 - z) * n + z * x


def _gru_step_body(x_ref, agg_ref, wih_ref, whh_ref, bih_ref, bhh_ref,
                   wn_ref, xo_ref, m_ref):
    xn = _gru_math(x_ref, agg_ref, wih_ref, whh_ref, bih_ref, bhh_ref)
    xo_ref[0] = xn
    m = jnp.dot(xn, wn_ref[0], preferred_element_type=jnp.float32)
    m_ref[0, 0] = m[:, :HC]
    m_ref[1, 0] = m[:, HC:]


def _gru_last_body(x_ref, agg_ref, wih_ref, whh_ref, bih_ref, bhh_ref,
                   xo_ref):
    xo_ref[0] = _gru_math(x_ref, agg_ref, wih_ref, whh_ref, bih_ref, bhh_ref)


_GRU_IN_SPECS = [
    pl.BlockSpec((1, BR, C), lambda t, rb: (t, rb, 0)),
    pl.BlockSpec((2, 1, BR, HC), lambda t, rb: (0, t, rb, 0)),
    pl.BlockSpec((1, 3 * C, C), lambda t, rb: (t, 0, 0)),
    pl.BlockSpec((1, 3 * C, C), lambda t, rb: (t, 0, 0)),
    pl.BlockSpec((1, 1, 3 * C), lambda t, rb: (t, 0, 0)),
    pl.BlockSpec((1, 1, 3 * C), lambda t, rb: (t, 0, 0)),
]


def _gru_step(x_all, agg, wih, whh, bih, bhh, wn):
    return pl.pallas_call(
        _gru_step_body,
        grid=(T, NB),
        in_specs=_GRU_IN_SPECS + [pl.BlockSpec((1, C, C), lambda t, rb: (t, 0, 0))],
        out_specs=[
            pl.BlockSpec((1, BR, C), lambda t, rb: (t, rb, 0)),
            pl.BlockSpec((2, 1, BR, HC), lambda t, rb: (0, t, rb, 0)),
        ],
        out_shape=[
            jax.ShapeDtypeStruct((T, N, C), jnp.float32),
            jax.ShapeDtypeStruct((2, T, N, HC), jnp.float32),
        ],
    )(x_all, agg, wih, whh, bih, bhh, wn)


def _gru_last(x_all, agg, wih, whh, bih, bhh):
    return pl.pallas_call(
        _gru_last_body,
        grid=(T, NB),
        in_specs=_GRU_IN_SPECS,
        out_specs=pl.BlockSpec((1, BR, C), lambda t, rb: (t, rb, 0)),
        out_shape=jax.ShapeDtypeStruct((T, N, C), jnp.float32),
    )(x_all, agg, wih, whh, bih, bhh)


def _blockend_body(h_ref, x_ref, g_ref, b_ref, o_ref):
    hh = h_ref[:] + x_ref[0] + x_ref[1] + x_ref[2]
    mu = jnp.mean(hh, axis=-1, keepdims=True)
    var = jnp.mean((hh - mu) ** 2, axis=-1, keepdims=True)
    hn = (hh - mu) * lax.rsqrt(var + EPS) * g_ref[:] + b_ref[:]
    o_ref[:] = jnp.maximum(hn, 0.0)


def _blockend(h, x_all, gamma, beta):
    return pl.pallas_call(
        _blockend_body,
        out_shape=jax.ShapeDtypeStruct((N, C), jnp.float32),
    )(h, x_all, gamma, beta)


def _head_body(h_ref, w1_ref, b1_ref, w2_ref, b2_ref, o_ref):
    hid = jnp.maximum(
        jnp.dot(h_ref[:], w1_ref[:], preferred_element_type=jnp.float32)
        + b1_ref[:], 0.0)
    o_ref[:] = jnp.dot(hid, w2_ref[:], preferred_element_type=jnp.float32) + b2_ref[:]


def _head(h, w1, b1, w2p, b2p):
    return pl.pallas_call(
        _head_body,
        out_shape=jax.ShapeDtypeStruct((N, C), jnp.float32),
    )(h, w1, b1, w2p, b2p)


def kernel(x_type, x_tok, x_small, edge_index, edge_type, batch,
           conv_weight, gru_wih, gru_whh, gru_bih, gru_bhh,
           ln_gamma, ln_beta, head_w1, head_b1, head_w2, head_b2):
    del batch

    h = _feats(x_type, x_tok, x_small)
    eidx = _sc_prologue(edge_index.reshape(2 * E), edge_type)
    zeros_tbl = jnp.zeros((R3N, HC), jnp.float32)

    bih3 = gru_bih.reshape(BLOCKS, T, 1, 3 * C)
    bhh3 = gru_bhh.reshape(BLOCKS, T, 1, 3 * C)

    for b in range(BLOCKS):
        x_all, m = _conv0(h, conv_weight[b, :, 0])
        for s in range(STEPS):
            agg = _sc_scatter(m.reshape(2 * R3N, HC), eidx, zeros_tbl)
            agg = agg.reshape(2, T, N, HC)
            if s < STEPS - 1:
                x_all, m = _gru_step(x_all, agg, gru_wih[b], gru_whh[b],
                                     bih3[b], bhh3[b], conv_weight[b, :, s + 1])
            else:
                x_all = _gru_last(x_all, agg, gru_wih[b], gru_whh[b],
                                  bih3[b], bhh3[b])
        h = _blockend(h, x_all, ln_gamma[b].reshape(1, C),
                      ln_beta[b].reshape(1, C))

    w2p = jnp.zeros((C, C), jnp.float32).at[:, :2].set(head_w2)
    b2p = jnp.zeros((1, C), jnp.float32).at[0, :2].set(head_b2)
    out = _head(h, head_w1, head_b1.reshape(1, C), w2p, b2p)
    return out[:, :2]


# speedup vs baseline: 1.6937x; 1.0031x over previous
import functools

import jax
import jax.numpy as jnp
from jax import lax
from jax.experimental import pallas as pl
from jax.experimental.pallas import tpu as pltpu
from jax.experimental.pallas import tpu_sc as plsc

N = 10000
C = 128
HC = 64
E = 320000
NUM_TYPES = 64
DIM_TOK = 62
STEPS = 4
BLOCKS = 2
T = 3
EPS = 1e-5
BR = 2000
NB = N // BR
R3N = 3 * N

K = 80
CHUNKS = E // K
NTILES = 16
CPT = CHUNKS // NTILES
EPT = E // NTILES

def _mesh():
    return plsc.VectorSubcoreMesh(core_axis_name="c", subcore_axis_name="s")


def _sc_prologue_body(ei_hbm, et_hbm, idx_hbm, srcb, dstb, etb, obuf):
    c = lax.axis_index("c")
    tid = lax.axis_index("s")
    c3n = c * R3N
    base = tid * EPT
    pltpu.sync_copy(ei_hbm.at[pl.ds(base, EPT)], srcb)
    pltpu.sync_copy(ei_hbm.at[pl.ds(E + base, EPT)], dstb)
    pltpu.sync_copy(et_hbm.at[pl.ds(base, EPT)], etb)

    def body(i, _):
        for l in range(K // 16):
            off = i * K + l * 16
            s = srcb[pl.ds(off, 16)]
            d = dstb[pl.ds(off, 16)]
            t = etb[pl.ds(off, 16)]
            tn = t * N
            obuf[i, 0, pl.ds(l * 16, 16)] = tn + s + c3n
            obuf[i, 1, pl.ds(l * 16, 16)] = tn + d
        return 0

    lax.fori_loop(0, CPT, body, 0)
    pltpu.sync_copy(obuf, idx_hbm.at[c, tid])


def _sc_prologue(edge_index_flat, edge_type):
    f = pl.kernel(
        _sc_prologue_body,
        out_type=jax.ShapeDtypeStruct((2, NTILES, CPT, 2, K), jnp.int32),
        mesh=_mesh(),
        scratch_types=[
            pltpu.VMEM((EPT,), jnp.int32),
            pltpu.VMEM((EPT,), jnp.int32),
            pltpu.VMEM((EPT,), jnp.int32),
            pltpu.VMEM((CPT, 2, K), jnp.int32),
        ],
        compiler_params=pltpu.CompilerParams(use_tc_tiling_on_sc=False),
    )
    return f(edge_index_flat, edge_type)


def _sc_scatter_body(m2_hbm, idx_hbm, z_hbm, agg_hbm,
                     idxb, rows, acc, semi0, semi1, semi2, semi3, semg0, semg1,
                     semg2a, semg2b):
    c = lax.axis_index("c")
    tid = lax.axis_index("s")
    semi = (semi0, semi1, semi2, semi3)
    semg = (semg0, semg1)
    semg2 = (semg2a, semg2b)
    rpt = 1880
    rlast = R3N - 15 * rpt

    def fire_idx(j, q):
        pltpu.async_copy(idx_hbm.at[c, tid, j], idxb.at[q], semi[q])

    def wait_idx(j, q):
        pltpu.make_async_copy(idx_hbm.at[c, tid, j], idxb.at[q], semi[q]).wait()

    def fire_gather(j, q, r):
        pltpu.async_copy(m2_hbm.at[idxb.at[q, 0, pl.ds(0, K // 2)]],
                         rows.at[r, pl.ds(0, K // 2)], semg[r])
        pltpu.async_copy(m2_hbm.at[idxb.at[q, 0, pl.ds(K // 2, K // 2)]],
                         rows.at[r, pl.ds(K // 2, K // 2)], semg2[r])

    def wait_gather(r):
        pltpu.make_async_copy(m2_hbm.at[idxb.at[0, 0, pl.ds(0, K // 2)]],
                              rows.at[r, pl.ds(0, K // 2)], semg[r]).wait()
        pltpu.make_async_copy(m2_hbm.at[idxb.at[0, 0, pl.ds(0, K // 2)]],
                              rows.at[r, pl.ds(K // 2, K // 2)], semg2[r]).wait()

    for q in range(4):
        fire_idx(q, q)
    wait_idx(0, 0)
    fire_gather(0, 0, 0)
    wait_idx(1, 1)
    fire_gather(1, 1, 1)

    @pl.when(tid < NTILES - 1)
    def _():
        pltpu.sync_copy(z_hbm.at[pl.ds(tid * rpt, rpt)],
                        acc.at[pl.ds(tid * rpt, rpt)])

    @pl.when(tid == NTILES - 1)
    def _():
        pltpu.sync_copy(z_hbm.at[pl.ds(15 * rpt, rlast)],
                        acc.at[pl.ds(15 * rpt, rlast)])

    plsc.subcore_barrier()

    def body(i, _):
        for k in range(4):
            j = 4 * i + k
            r = k % 2
            wait_gather(r)
            pltpu.sync_copy(rows.at[r], acc.at[idxb.at[k, 1]], add=True)

            @pl.when(j + 4 < CPT)
            def _():
                fire_idx(j + 4, k)

            @pl.when(j + 2 < CPT)
            def _():
                wait_idx(j + 2, (k + 2) % 4)
                fire_gather(j + 2, (k + 2) % 4, r)

        return 0

    lax.fori_loop(0, CPT // 4, body, 0)
    wait_gather(0)
    pltpu.sync_copy(rows.at[0], acc.at[idxb.at[0, 1]], add=True)
    wait_gather(1)
    pltpu.sync_copy(rows.at[1], acc.at[idxb.at[1, 1]], add=True)

    plsc.subcore_barrier()

    @pl.when(tid < NTILES - 1)
    def _():
        pltpu.sync_copy(acc.at[pl.ds(tid * rpt, rpt)],
                        agg_hbm.at[pl.ds(c * R3N + tid * rpt, rpt)])

    @pl.when(tid == NTILES - 1)
    def _():
        pltpu.sync_copy(acc.at[pl.ds(15 * rpt, rlast)],
                        agg_hbm.at[pl.ds(c * R3N + 15 * rpt, rlast)])


def _sc_scatter(m2, idx, zeros_tbl):
    f = pl.kernel(
        _sc_scatter_body,
        out_type=jax.ShapeDtypeStruct((2 * R3N, HC), jnp.float32),
        mesh=_mesh(),
        scratch_types=[
            pltpu.VMEM((4, 2, K), jnp.int32),
            pltpu.VMEM((2, K, HC), jnp.float32),
            pltpu.VMEM_SHARED((R3N, HC), jnp.float32),
            pltpu.SemaphoreType.DMA,
            pltpu.SemaphoreType.DMA,
            pltpu.SemaphoreType.DMA,
            pltpu.SemaphoreType.DMA,
            pltpu.SemaphoreType.DMA,
            pltpu.SemaphoreType.DMA,
            pltpu.SemaphoreType.DMA,
            pltpu.SemaphoreType.DMA,
        ],
        compiler_params=pltpu.CompilerParams(use_tc_tiling_on_sc=False),
    )
    return f(m2, idx, zeros_tbl)


def _feats_body(xt_ref, xk_ref, xs_ref, h_ref):
    col = lax.broadcasted_iota(jnp.int32, (N, C), 1)
    t = xt_ref[:]
    k = jnp.clip(xk_ref[:], 0, DIM_TOK - 1)
    xs = xs_ref[:]
    h = jnp.where(col < NUM_TYPES, (col == t).astype(jnp.float32),
                  jnp.where(col < NUM_TYPES + DIM_TOK,
                            (col - NUM_TYPES == k).astype(jnp.float32), 0.0))
    h = jnp.where(col == C - 2, xs[:, 0:1], h)
    h = jnp.where(col == C - 1, xs[:, 1:2], h)
    h_ref[:] = h


def _feats(x_type, x_tok, x_small):
    return pl.pallas_call(
        _feats_body,
        out_shape=jax.ShapeDtypeStruct((N, C), jnp.float32),
    )(x_type, x_tok, x_small)


def _conv0_body(h_ref, w_ref, x_ref, m_ref):
    x = h_ref[:]
    x_ref[0] = x
    m = jnp.dot(x, w_ref[0], preferred_element_type=jnp.float32)
    m_ref[0, 0] = m[:, :HC]
    m_ref[1, 0] = m[:, HC:]


def _conv0(h, w):
    return pl.pallas_call(
        _conv0_body,
        grid=(T, NB),
        in_specs=[
            pl.BlockSpec((BR, C), lambda t, rb: (rb, 0)),
            pl.BlockSpec((1, C, C), lambda t, rb: (t, 0, 0)),
        ],
        out_specs=[
            pl.BlockSpec((1, BR, C), lambda t, rb: (t, rb, 0)),
            pl.BlockSpec((2, 1, BR, HC), lambda t, rb: (0, t, rb, 0)),
        ],
        out_shape=[
            jax.ShapeDtypeStruct((T, N, C), jnp.float32),
            jax.ShapeDtypeStruct((2, T, N, HC), jnp.float32),
        ],
    )(h, w)


def _gru_math(x_ref, agg_ref, wih_ref, whh_ref, bih_ref, bhh_ref):
    x = x_ref[0]
    a = jnp.concatenate([agg_ref[0, 0], agg_ref[1, 0]], axis=1)
    gi = lax.dot_general(a, wih_ref[0], (((1,), (1,)), ((), ())),
                         preferred_element_type=jnp.float32) + bih_ref[0]
    gh = lax.dot_general(x, whh_ref[0], (((1,), (1,)), ((), ())),
                         preferred_element_type=jnp.float32) + bhh_ref[0]
    r = jax.nn.sigmoid(gi[:, :C] + gh[:, :C])
    z = jax.nn.sigmoid(gi[:, C:2 * C] + gh[:, C:2 * C])
    n = jnp.tanh(gi[:, 2 * C:] + r * gh[:, 2 * C:])
    return (1.0 - z) * n + z * x


def _gru_step_body(x_ref, agg_ref, wih_ref, whh_ref, bih_ref, bhh_ref,
                   wn_ref, xo_ref, m_ref):
    xn = _gru_math(x_ref, agg_ref, wih_ref, whh_ref, bih_ref, bhh_ref)
    xo_ref[0] = xn
    m = jnp.dot(xn, wn_ref[0], preferred_element_type=jnp.float32)
    m_ref[0, 0] = m[:, :HC]
    m_ref[1, 0] = m[:, HC:]


def _gru_last_body(x_ref, agg_ref, wih_ref, whh_ref, bih_ref, bhh_ref,
                   xo_ref):
    xo_ref[0] = _gru_math(x_ref, agg_ref, wih_ref, whh_ref, bih_ref, bhh_ref)


_GRU_IN_SPECS = [
    pl.BlockSpec((1, BR, C), lambda t, rb: (t, rb, 0)),
    pl.BlockSpec((2, 1, BR, HC), lambda t, rb: (0, t, rb, 0)),
    pl.BlockSpec((1, 3 * C, C), lambda t, rb: (t, 0, 0)),
    pl.BlockSpec((1, 3 * C, C), lambda t, rb: (t, 0, 0)),
    pl.BlockSpec((1, 1, 3 * C), lambda t, rb: (t, 0, 0)),
    pl.BlockSpec((1, 1, 3 * C), lambda t, rb: (t, 0, 0)),
]


def _gru_step(x_all, agg, wih, whh, bih, bhh, wn):
    return pl.pallas_call(
        _gru_step_body,
        grid=(T, NB),
        in_specs=_GRU_IN_SPECS + [pl.BlockSpec((1, C, C), lambda t, rb: (t, 0, 0))],
        out_specs=[
            pl.BlockSpec((1, BR, C), lambda t, rb: (t, rb, 0)),
            pl.BlockSpec((2, 1, BR, HC), lambda t, rb: (0, t, rb, 0)),
        ],
        out_shape=[
            jax.ShapeDtypeStruct((T, N, C), jnp.float32),
            jax.ShapeDtypeStruct((2, T, N, HC), jnp.float32),
        ],
    )(x_all, agg, wih, whh, bih, bhh, wn)


def _gru_last(x_all, agg, wih, whh, bih, bhh):
    return pl.pallas_call(
        _gru_last_body,
        grid=(T, NB),
        in_specs=_GRU_IN_SPECS,
        out_specs=pl.BlockSpec((1, BR, C), lambda t, rb: (t, rb, 0)),
        out_shape=jax.ShapeDtypeStruct((T, N, C), jnp.float32),
    )(x_all, agg, wih, whh, bih, bhh)


def _blockend_body(h_ref, x_ref, g_ref, b_ref, o_ref):
    hh = h_ref[:] + x_ref[0] + x_ref[1] + x_ref[2]
    mu = jnp.mean(hh, axis=-1, keepdims=True)
    var = jnp.mean((hh - mu) ** 2, axis=-1, keepdims=True)
    hn = (hh - mu) * lax.rsqrt(var + EPS) * g_ref[:] + b_ref[:]
    o_ref[:] = jnp.maximum(hn, 0.0)


def _blockend(h, x_all, gamma, beta):
    return pl.pallas_call(
        _blockend_body,
        out_shape=jax.ShapeDtypeStruct((N, C), jnp.float32),
    )(h, x_all, gamma, beta)


def _head_body(h_ref, w1_ref, b1_ref, w2_ref, b2_ref, o_ref):
    hid = jnp.maximum(
        jnp.dot(h_ref[:], w1_ref[:], preferred_element_type=jnp.float32)
        + b1_ref[:], 0.0)
    o_ref[:] = jnp.dot(hid, w2_ref[:], preferred_element_type=jnp.float32) + b2_ref[:]


def _head(h, w1, b1, w2p, b2p):
    return pl.pallas_call(
        _head_body,
        out_shape=jax.ShapeDtypeStruct((N, C), jnp.float32),
    )(h, w1, b1, w2p, b2p)


def kernel(x_type, x_tok, x_small, edge_index, edge_type, batch,
           conv_weight, gru_wih, gru_whh, gru_bih, gru_bhh,
           ln_gamma, ln_beta, head_w1, head_b1, head_w2, head_b2):
    del batch

    h = _feats(x_type, x_tok, x_small)
    eidx = _sc_prologue(edge_index.reshape(2 * E), edge_type)
    zeros_tbl = jnp.zeros((R3N, HC), jnp.float32)

    bih3 = gru_bih.reshape(BLOCKS, T, 1, 3 * C)
    bhh3 = gru_bhh.reshape(BLOCKS, T, 1, 3 * C)

    for b in range(BLOCKS):
        x_all, m = _conv0(h, conv_weight[b, :, 0])
        for s in range(STEPS):
            agg = _sc_scatter(m.reshape(2 * R3N, HC), eidx, zeros_tbl)
            agg = agg.reshape(2, T, N, HC)
            if s < STEPS - 1:
                x_all, m = _gru_step(x_all, agg, gru_wih[b], gru_whh[b],
                                     bih3[b], bhh3[b], conv_weight[b, :, s + 1])
            else:
                x_all = _gru_last(x_all, agg, gru_wih[b], gru_whh[b],
                                  bih3[b], bhh3[b])
        h = _blockend(h, x_all, ln_gamma[b].reshape(1, C),
                      ln_beta[b].reshape(1, C))

    w2p = jnp.zeros((C, C), jnp.float32).at[:, :2].set(head_w2)
    b2p = jnp.zeros((1, C), jnp.float32).at[0, :2].set(head_b2)
    out = _head(h, head_w1, head_b1.reshape(1, C), w2p, b2p)
    return out[:, :2]
